# baseline (speedup 1.0000x reference)
import functools
import jax
import jax.numpy as jnp
from jax import lax
from jax.experimental import pallas as pl
from jax.experimental.pallas import tpu as pltpu
from jax.experimental.pallas import tpu_sc as plsc

N = 4096
A_TAIL = 1536
A_HEAD = N - A_TAIL

NC = 2
NS = 16
NW = NC * NS
AT_W = N // NW
ATAIL_W = A_TAIL // NW
RB = 8
CHUNKS = N // 16
UNROLL = 4


def _compute_block(buf, v_ref, red_v, out_v, blk):
    zero = jnp.zeros((16,), jnp.float32)

    def jbody(j, accs):
        accs = list(accs)
        for u in range(UNROLL):
            sl = pl.ds((j * UNROLL + u) * 16, 16)
            vc = v_ref[sl]
            for r in range(RB):
                accs[r] = accs[r] + buf[r, sl] * vc
        return tuple(accs)

    accs = lax.fori_loop(0, CHUNKS // UNROLL, jbody, (zero,) * RB)
    for r in range(RB):
        red_v[r, :] = accs[r]
    lanes = lax.iota(jnp.int32, 16)
    row_idx = lanes & (RB - 1)
    hsum = zero
    for j in range(16):
        hsum = hsum + plsc.load_gather(
            red_v, [row_idx, jnp.full((16,), j, jnp.int32)])
    plsc.store_scatter(out_v, [blk * RB + row_idx], hsum, mask=lanes < RB)


@functools.cache
def _get_sc_matvecs():
  mesh = plsc.VectorSubcoreMesh(core_axis_name="c", subcore_axis_name="s",
                                num_cores=NC, num_subcores=NS)

  @functools.partial(
      pl.kernel,
      out_type=(jax.ShapeDtypeStruct((N,), jnp.float32),
                jax.ShapeDtypeStruct((A_TAIL,), jnp.float32)),
      mesh=mesh,
      compiler_params=pltpu.CompilerParams(needs_layout_passes=False),
      scratch_types=[
          pltpu.VMEM((N,), jnp.float32),
          pltpu.VMEM((N,), jnp.float32),
          pltpu.VMEM((N,), jnp.float32),
          pltpu.VMEM((RB, N), jnp.float32),
          pltpu.VMEM((RB, N), jnp.float32),
          pltpu.VMEM((AT_W,), jnp.float32),
          pltpu.VMEM((ATAIL_W,), jnp.float32),
          pltpu.VMEM((RB, 16), jnp.float32),
          pltpu.VMEM((16,), jnp.float32),
          pltpu.SemaphoreType.DMA,
          pltpu.SemaphoreType.DMA,
      ],
  )
  def _sc_matvecs(AT_hbm, A_hbm, x_hbm, y_hbm, vs_hbm, cs_hbm, cons_hbm,
                  aty_hbm, axt_hbm,
                  xun_v, yun_v, tmp_v, buf0, buf1, atyv, axtv, red_v, cons_v,
                  sem0, sem1):
    c = lax.axis_index("c")
    s = lax.axis_index("s")
    wid = s * NC + c

    pltpu.sync_copy(cons_hbm, cons_v)
    csv = cons_v[...]

    pltpu.sync_copy(x_hbm, xun_v)
    pltpu.sync_copy(vs_hbm, tmp_v)

    def unscale_x(j, _):
        sl = pl.ds(j * 16, 16)
        xun_v[sl] = xun_v[sl] / tmp_v[sl] * csv
        return 0

    lax.fori_loop(0, CHUNKS, unscale_x, 0)

    pltpu.sync_copy(y_hbm, yun_v)
    pltpu.sync_copy(cs_hbm, tmp_v)

    def unscale_y(j, _):
        sl = pl.ds(j * 16, 16)
        yun_v[sl] = yun_v[sl] / tmp_v[sl] * csv
        return 0

    lax.fori_loop(0, CHUNKS, unscale_y, 0)

    def do_matvec(M_hbm, row0, nrows, v_ref, out_v):
        nblk = nrows // RB
        pltpu.async_copy(M_hbm.at[pl.ds(row0, RB)], buf0, sem0)

        def outer(k, _):
            blk0 = k * 2
            pltpu.async_copy(
                M_hbm.at[pl.ds(row0 + (blk0 + 1) * RB, RB)], buf1, sem1)
            pltpu.make_async_copy(
                M_hbm.at[pl.ds(row0, RB)], buf0, sem0).wait()
            _compute_block(buf0, v_ref, red_v, out_v, blk0)

            @pl.when(blk0 + 2 < nblk)
            def _():
                pltpu.async_copy(
                    M_hbm.at[pl.ds(row0 + (blk0 + 2) * RB, RB)], buf0, sem0)

            pltpu.make_async_copy(
                M_hbm.at[pl.ds(row0, RB)], buf1, sem1).wait()
            _compute_block(buf1, v_ref, red_v, out_v, blk0 + 1)
            return 0

        lax.fori_loop(0, nblk // 2, outer, 0)

    do_matvec(AT_hbm, wid * AT_W, AT_W, yun_v, atyv)
    do_matvec(A_hbm, A_HEAD + wid * ATAIL_W, ATAIL_W, xun_v, axtv)

    pltpu.sync_copy(atyv, aty_hbm.at[pl.ds(wid * AT_W, AT_W)])
    pltpu.sync_copy(axtv, axt_hbm.at[pl.ds(wid * ATAIL_W, ATAIL_W)])

  return _sc_matvecs


BLK = 512
GRID = N // BLK
R2 = BLK // 128

_VAR, _B, _QX, _C, _QUAD, _LIN, _VIOT = range(7)


def _tcq_kernel(cons_ref, Q_ref, xc_ref, vsc_ref,
                x2_ref, vs2_ref, y2_ref, cs2_ref, b2_ref, c2_ref,
                il2_ref, iu2_ref, l2_ref, u2_ref,
                qx2_out, part_ref, acc_ref):
    i = pl.program_id(0)
    relu = jax.nn.relu
    cs = cons_ref[0]

    xun_c = xc_ref[...] / vsc_ref[...] * cs
    Qx2 = jnp.dot(Q_ref[...], xun_c,
                  preferred_element_type=jnp.float32).reshape(1, R2, 128)
    qx2_out[...] = Qx2

    xun2 = x2_ref[...] / vs2_ref[...] * cs
    yun2 = y2_ref[...] / cs2_ref[...] * cs
    b2 = b2_ref[...]
    c2 = c2_ref[...]

    var_vio = relu(l2_ref[...] - xun2) * il2_ref[...] + \
        relu(xun2 - u2_ref[...]) * iu2_ref[...]

    p_var = jnp.max(jnp.abs(var_vio))
    p_b = jnp.max(jnp.abs(b2))
    p_qx = jnp.max(jnp.abs(Qx2))
    p_c = jnp.max(jnp.abs(c2))
    s_quad = jnp.sum(xun2 * Qx2)
    s_lin = jnp.sum(c2 * xun2)
    s_vio = jnp.sum(b2 * yun2)

    @pl.when(i == 0)
    def _init():
        for k in range(7):
            acc_ref[k] = 0.0

    acc_ref[_VAR] = jnp.maximum(acc_ref[_VAR], p_var)
    acc_ref[_B] = jnp.maximum(acc_ref[_B], p_b)
    acc_ref[_QX] = jnp.maximum(acc_ref[_QX], p_qx)
    acc_ref[_C] = jnp.maximum(acc_ref[_C], p_c)
    acc_ref[_QUAD] = acc_ref[_QUAD] + s_quad
    acc_ref[_LIN] = acc_ref[_LIN] + s_lin
    acc_ref[_VIOT] = acc_ref[_VIOT] + s_vio

    @pl.when(i == GRID - 1)
    def _fin():
        for k in range(7):
            part_ref[k] = acc_ref[k]


AGRID = A_HEAD // BLK


def _tca_kernel(cons_ref, A_ref, xc_ref, vsc_ref, b2_ref, Iy2_ref,
                part_ref, acc_ref):
    i = pl.program_id(0)
    relu = jax.nn.relu
    cs = cons_ref[0]

    xun_c = xc_ref[...] / vsc_ref[...] * cs
    Ax2 = jnp.dot(A_ref[...], xun_c,
                  preferred_element_type=jnp.float32).reshape(1, R2, 128)
    cv = b2_ref[...] - Ax2
    cv = cv + relu(-cv) * Iy2_ref[...]

    p_cv = jnp.max(jnp.abs(cv))
    p_ax = jnp.max(jnp.abs(Ax2))

    @pl.when(i == 0)
    def _init():
        acc_ref[0] = 0.0
        acc_ref[1] = 0.0

    acc_ref[0] = jnp.maximum(acc_ref[0], p_cv)
    acc_ref[1] = jnp.maximum(acc_ref[1], p_ax)

    @pl.when(i == AGRID - 1)
    def _fin():
        part_ref[0] = acc_ref[0]
        part_ref[1] = acc_ref[1]


def _tc3_kernel(cons_ref, pq_ref, pa_ref,
                qx2_ref, aty2_ref, axt2_ref, bt2_ref, Iyt2_ref,
                c2_ref, y2_ref, cs2_ref, Iy2_ref,
                il2_ref, iu2_ref, l2_ref, u2_ref, out_ref):
    relu = jax.nn.relu
    cs = cons_ref[0]
    yun = y2_ref[...] / cs2_ref[...] * cs

    Qx = qx2_ref[...]
    ATy = aty2_ref[...]
    c = c2_ref[...]

    pg = c - ATy + Qx
    rpg = relu(pg)
    rng = relu(-pg)
    il = il2_ref[...]
    iu = iu2_ref[...]
    RCV = pg - rpg * il + rng * iu
    DR = relu(-yun) * Iy2_ref[...]
    RC = rpg * il - rng * iu
    tm = jnp.where(RC > 0, l2_ref[...], u2_ref[...])

    cv_t = bt2_ref[...] - axt2_ref[...]
    cv_t = cv_t + relu(-cv_t) * Iyt2_ref[...]

    m_var = pq_ref[_VAR]
    m_cv = jnp.maximum(pa_ref[0], jnp.max(jnp.abs(cv_t)))
    m_ax = jnp.maximum(pa_ref[1], jnp.max(jnp.abs(axt2_ref[...])))
    t1 = jnp.maximum(m_var, m_cv) / (1.0 + jnp.maximum(m_ax, pq_ref[_B]))

    m_rcv = jnp.maximum(jnp.max(jnp.abs(RCV)), jnp.max(jnp.abs(DR)))
    m_aty = jnp.max(jnp.abs(ATy))
    t2 = m_rcv / (1.0 + jnp.maximum(pq_ref[_QX],
                                    jnp.maximum(m_aty, pq_ref[_C])))

    quad = pq_ref[_QUAD]
    lin = pq_ref[_LIN]
    vio = pq_ref[_VIOT]
    rcc = jnp.sum(RC * tm)
    t3 = jnp.abs(quad + lin - vio - rcc) / (
        1.0 + jnp.maximum(jnp.abs(vio - 0.5 * quad),
                          jnp.abs(0.5 * quad + lin)))
    res = jnp.maximum(t1, jnp.maximum(t2, t3))
    out_ref[0] = res
    out_ref[1] = t1
    out_ref[2] = t2
    out_ref[3] = t3


def kernel(Q, A, AT, b, c, x, y, Iy, il, iu, l, u, vscale, cscale, cons_scale):
    xf = x.reshape(N)
    yf = y.reshape(N)
    vsf = vscale.reshape(N)
    csf = cscale.reshape(N)
    cons1 = cons_scale.reshape(1)
    cons16 = jnp.broadcast_to(cons1, (16,))

    aty, ax_tail = _get_sc_matvecs()(AT, A, xf, yf, vsf, csf, cons16)

    sq = (32, 128)
    s3 = (GRID, R2, 128)
    x2 = x.reshape(s3)
    vs2 = vscale.reshape(s3)
    y2 = y.reshape(s3)
    cs2 = cscale.reshape(s3)
    b2 = b.reshape(s3)
    c2 = c.reshape(s3)
    Iy2 = Iy.reshape(s3)
    il2 = il.reshape(s3)
    iu2 = iu.reshape(s3)
    l2 = l.reshape(s3)
    u2 = u.reshape(s3)

    row_q = pl.BlockSpec((BLK, N), lambda i: (i, 0))
    v2 = pl.BlockSpec((1, R2, 128), lambda i: (i, 0, 0))
    full_col = pl.BlockSpec((N, 1), lambda i: (0, 0))
    smem = pl.BlockSpec(memory_space=pltpu.SMEM)

    qx2, parts_q = pl.pallas_call(
        _tcq_kernel,
        grid=(GRID,),
        in_specs=[smem, row_q, full_col, full_col] + [v2] * 10,
        out_specs=(v2, smem),
        out_shape=(jax.ShapeDtypeStruct(s3, jnp.float32),
                   jax.ShapeDtypeStruct((7,), jnp.float32)),
        scratch_shapes=[pltpu.SMEM((7,), jnp.float32)],
    )(cons1, Q, x, vscale, x2, vs2, y2, cs2, b2, c2, il2, iu2, l2, u2)

    parts_a = pl.pallas_call(
        _tca_kernel,
        grid=(AGRID,),
        in_specs=[smem, row_q, full_col, full_col, v2, v2],
        out_specs=smem,
        out_shape=jax.ShapeDtypeStruct((2,), jnp.float32),
        scratch_shapes=[pltpu.SMEM((2,), jnp.float32)],
    )(cons1, A, x, vscale, b2, Iy2)

    st = (A_TAIL // 128, 128)
    g0 = lambda: (0, 0)
    out = pl.pallas_call(
        _tc3_kernel,
        in_specs=[smem] * 3
        + [pl.BlockSpec(sq, g0)] * 2
        + [pl.BlockSpec(st, g0)] * 3
        + [pl.BlockSpec(sq, g0)] * 8,
        out_specs=smem,
        out_shape=jax.ShapeDtypeStruct((4,), jnp.float32),
    )(cons1, parts_q, parts_a,
      qx2.reshape(sq), aty.reshape(sq),
      ax_tail.reshape(st), b.reshape(sq)[A_HEAD // 128:],
      Iy.reshape(sq)[A_HEAD // 128:],
      c.reshape(sq), y.reshape(sq), cscale.reshape(sq), Iy.reshape(sq),
      il.reshape(sq), iu.reshape(sq), l.reshape(sq), u.reshape(sq))

    res = out[0].reshape(1, 1)
    t1 = out[1].reshape(())
    t2 = out[2].reshape(())
    t3 = out[3].reshape(1, 1)
    return res, t1, t2, t3


# speedup vs baseline: 1.0161x; 1.0161x over previous
import functools
import jax
import jax.numpy as jnp
from jax import lax
from jax.experimental import pallas as pl
from jax.experimental.pallas import tpu as pltpu
from jax.experimental.pallas import tpu_sc as plsc

N = 4096
A_TAIL = 1536
A_HEAD = N - A_TAIL

NC = 2
NS = 16
NW = NC * NS
AT_W = N // NW
ATAIL_W = A_TAIL // NW
RB = 8
CHUNKS = N // 16
UNROLL = 4


def _compute_block(buf, v_ref, red_v, out_v, blk):
    zero = jnp.zeros((16,), jnp.float32)

    def jbody(j, accs):
        accs = list(accs)
        for u in range(UNROLL):
            sl = pl.ds((j * UNROLL + u) * 16, 16)
            vc = v_ref[sl]
            for r in range(RB):
                accs[r] = accs[r] + buf[r, sl] * vc
        return tuple(accs)

    accs = lax.fori_loop(0, CHUNKS // UNROLL, jbody, (zero,) * RB)
    for r in range(RB):
        red_v[r, :] = accs[r]
    lanes = lax.iota(jnp.int32, 16)
    row_idx = lanes & (RB - 1)
    hsum = zero
    for j in range(16):
        hsum = hsum + plsc.load_gather(
            red_v, [row_idx, jnp.full((16,), j, jnp.int32)])
    plsc.store_scatter(out_v, [blk * RB + row_idx], hsum, mask=lanes < RB)


@functools.cache
def _get_sc_matvecs():
  mesh = plsc.VectorSubcoreMesh(core_axis_name="c", subcore_axis_name="s",
                                num_cores=NC, num_subcores=NS)

  @functools.partial(
      pl.kernel,
      out_type=(jax.ShapeDtypeStruct((N,), jnp.float32),
                jax.ShapeDtypeStruct((A_TAIL,), jnp.float32)),
      mesh=mesh,
      compiler_params=pltpu.CompilerParams(needs_layout_passes=False),
      scratch_types=[
          pltpu.VMEM((N,), jnp.float32),
          pltpu.VMEM((N,), jnp.float32),
          pltpu.VMEM((N,), jnp.float32),
          pltpu.VMEM((RB, N), jnp.float32),
          pltpu.VMEM((RB, N), jnp.float32),
          pltpu.VMEM((AT_W,), jnp.float32),
          pltpu.VMEM((ATAIL_W,), jnp.float32),
          pltpu.VMEM((RB, 16), jnp.float32),
          pltpu.VMEM((16,), jnp.float32),
          pltpu.SemaphoreType.DMA,
          pltpu.SemaphoreType.DMA,
      ],
  )
  def _sc_matvecs(AT_hbm, A_hbm, x_hbm, y_hbm, vs_hbm, cs_hbm, cons_hbm,
                  aty_hbm, axt_hbm,
                  xun_v, yun_v, tmp_v, buf0, buf1, atyv, axtv, red_v, cons_v,
                  sem0, sem1):
    c = lax.axis_index("c")
    s = lax.axis_index("s")
    wid = s * NC + c

    pltpu.sync_copy(cons_hbm, cons_v)
    csv = cons_v[...]

    pltpu.sync_copy(x_hbm, xun_v)
    pltpu.sync_copy(vs_hbm, tmp_v)

    def unscale_x(j, _):
        sl = pl.ds(j * 16, 16)
        xun_v[sl] = xun_v[sl] / tmp_v[sl] * csv
        return 0

    lax.fori_loop(0, CHUNKS, unscale_x, 0)

    pltpu.sync_copy(y_hbm, yun_v)
    pltpu.sync_copy(cs_hbm, tmp_v)

    def unscale_y(j, _):
        sl = pl.ds(j * 16, 16)
        yun_v[sl] = yun_v[sl] / tmp_v[sl] * csv
        return 0

    lax.fori_loop(0, CHUNKS, unscale_y, 0)

    def do_matvec(M_hbm, row0, nrows, v_ref, out_v):
        nblk = nrows // RB
        pltpu.async_copy(M_hbm.at[pl.ds(row0, RB)], buf0, sem0)

        def outer(k, _):
            blk0 = k * 2
            pltpu.async_copy(
                M_hbm.at[pl.ds(row0 + (blk0 + 1) * RB, RB)], buf1, sem1)
            pltpu.make_async_copy(
                M_hbm.at[pl.ds(row0, RB)], buf0, sem0).wait()
            _compute_block(buf0, v_ref, red_v, out_v, blk0)

            @pl.when(blk0 + 2 < nblk)
            def _():
                pltpu.async_copy(
                    M_hbm.at[pl.ds(row0 + (blk0 + 2) * RB, RB)], buf0, sem0)

            pltpu.make_async_copy(
                M_hbm.at[pl.ds(row0, RB)], buf1, sem1).wait()
            _compute_block(buf1, v_ref, red_v, out_v, blk0 + 1)
            return 0

        lax.fori_loop(0, nblk // 2, outer, 0)

    do_matvec(AT_hbm, wid * AT_W, AT_W, yun_v, atyv)
    do_matvec(A_hbm, A_HEAD + wid * ATAIL_W, ATAIL_W, xun_v, axtv)

    pltpu.sync_copy(atyv, aty_hbm.at[pl.ds(wid * AT_W, AT_W)])
    pltpu.sync_copy(axtv, axt_hbm.at[pl.ds(wid * ATAIL_W, ATAIL_W)])

  return _sc_matvecs


BLK = 256
NB_Q = N // BLK
NB_A = A_HEAD // BLK
NB = NB_Q + NB_A
DEPTH = 4
R2 = BLK // 128

_VAR, _B, _QX, _C, _QUAD, _LIN, _VIOT, _CVH, _AXH = range(9)


def _tcs_kernel(cons_ref, Q_ref, A_ref, xc_ref, vsc_ref,
                x2_ref, vs2_ref, y2_ref, cs2_ref, b2_ref, c2_ref,
                Iy2_ref, il2_ref, iu2_ref, l2_ref, u2_ref,
                qx2_out, part_ref, qbuf, acc_ref, sems):
    relu = jax.nn.relu
    cs = cons_ref[0]
    xun_c = xc_ref[...] / vsc_ref[...] * cs

    def src_of(i):
        qrow = i * BLK
        arow = jnp.maximum(i - NB_Q, 0) * BLK
        return qrow, arow

    def start_dma(i, slot):
        qrow, arow = src_of(i)

        @pl.when(i < NB_Q)
        def _():
            pltpu.make_async_copy(
                Q_ref.at[pl.ds(qrow, BLK)], qbuf.at[slot], sems.at[slot]
            ).start()

        @pl.when(i >= NB_Q)
        def _():
            pltpu.make_async_copy(
                A_ref.at[pl.ds(arow, BLK)], qbuf.at[slot], sems.at[slot]
            ).start()

    for d in range(DEPTH):
        start_dma(d, d)

    for k in range(9):
        acc_ref[k] = 0.0

    def step(i, _):
        slot = lax.rem(i, DEPTH)
        pltpu.make_async_copy(
            Q_ref.at[pl.ds(0, BLK)], qbuf.at[slot], sems.at[slot]
        ).wait()
        mb = qbuf[pl.ds(slot, 1)].reshape(BLK, N)
        mv2 = jnp.dot(mb, xun_c,
                      preferred_element_type=jnp.float32).reshape(R2, 128)

        sl = pl.ds(lax.rem(i, NB_Q) * R2, R2)

        @pl.when(i < NB_Q)
        def _qpart():
            qx2_out[sl, :] = mv2
            xun2 = x2_ref[sl, :] / vs2_ref[sl, :] * cs
            yun2 = y2_ref[sl, :] / cs2_ref[sl, :] * cs
            b2 = b2_ref[sl, :]
            c2 = c2_ref[sl, :]
            var_vio = relu(l2_ref[sl, :] - xun2) * il2_ref[sl, :] + \
                relu(xun2 - u2_ref[sl, :]) * iu2_ref[sl, :]
            acc_ref[_VAR] = jnp.maximum(acc_ref[_VAR],
                                        jnp.max(jnp.abs(var_vio)))
            acc_ref[_B] = jnp.maximum(acc_ref[_B], jnp.max(jnp.abs(b2)))
            acc_ref[_QX] = jnp.maximum(acc_ref[_QX], jnp.max(jnp.abs(mv2)))
            acc_ref[_C] = jnp.maximum(acc_ref[_C], jnp.max(jnp.abs(c2)))
            acc_ref[_QUAD] = acc_ref[_QUAD] + jnp.sum(xun2 * mv2)
            acc_ref[_LIN] = acc_ref[_LIN] + jnp.sum(c2 * xun2)
            acc_ref[_VIOT] = acc_ref[_VIOT] + jnp.sum(b2 * yun2)

        @pl.when(i >= NB_Q)
        def _apart():
            asl = pl.ds(jnp.maximum(i - NB_Q, 0) * R2, R2)
            cv = b2_ref[asl, :] - mv2
            cv = cv + relu(-cv) * Iy2_ref[asl, :]
            acc_ref[_CVH] = jnp.maximum(acc_ref[_CVH],
                                        jnp.max(jnp.abs(cv)))
            acc_ref[_AXH] = jnp.maximum(acc_ref[_AXH],
                                        jnp.max(jnp.abs(mv2)))

        @pl.when(i + DEPTH < NB)
        def _():
            start_dma(i + DEPTH, slot)

        return 0

    lax.fori_loop(0, NB, step, 0)

    for k in range(9):
        part_ref[k] = acc_ref[k]


def _tc3_kernel(cons_ref, pq_ref,
                qx2_ref, aty2_ref, axt2_ref, bt2_ref, Iyt2_ref,
                c2_ref, y2_ref, cs2_ref, Iy2_ref,
                il2_ref, iu2_ref, l2_ref, u2_ref, out_ref):
    relu = jax.nn.relu
    cs = cons_ref[0]
    yun = y2_ref[...] / cs2_ref[...] * cs

    Qx = qx2_ref[...]
    ATy = aty2_ref[...]
    c = c2_ref[...]

    pg = c - ATy + Qx
    rpg = relu(pg)
    rng = relu(-pg)
    il = il2_ref[...]
    iu = iu2_ref[...]
    RCV = pg - rpg * il + rng * iu
    DR = relu(-yun) * Iy2_ref[...]
    RC = rpg * il - rng * iu
    tm = jnp.where(RC > 0, l2_ref[...], u2_ref[...])

    cv_t = bt2_ref[...] - axt2_ref[...]
    cv_t = cv_t + relu(-cv_t) * Iyt2_ref[...]

    m_var = pq_ref[_VAR]
    m_cv = jnp.maximum(pq_ref[_CVH], jnp.max(jnp.abs(cv_t)))
    m_ax = jnp.maximum(pq_ref[_AXH], jnp.max(jnp.abs(axt2_ref[...])))
    t1 = jnp.maximum(m_var, m_cv) / (1.0 + jnp.maximum(m_ax, pq_ref[_B]))

    m_rcv = jnp.maximum(jnp.max(jnp.abs(RCV)), jnp.max(jnp.abs(DR)))
    m_aty = jnp.max(jnp.abs(ATy))
    t2 = m_rcv / (1.0 + jnp.maximum(pq_ref[_QX],
                                    jnp.maximum(m_aty, pq_ref[_C])))

    quad = pq_ref[_QUAD]
    lin = pq_ref[_LIN]
    vio = pq_ref[_VIOT]
    rcc = jnp.sum(RC * tm)
    t3 = jnp.abs(quad + lin - vio - rcc) / (
        1.0 + jnp.maximum(jnp.abs(vio - 0.5 * quad),
                          jnp.abs(0.5 * quad + lin)))
    res = jnp.maximum(t1, jnp.maximum(t2, t3))
    out_ref[0] = res
    out_ref[1] = t1
    out_ref[2] = t2
    out_ref[3] = t3


def kernel(Q, A, AT, b, c, x, y, Iy, il, iu, l, u, vscale, cscale, cons_scale):
    xf = x.reshape(N)
    yf = y.reshape(N)
    vsf = vscale.reshape(N)
    csf = cscale.reshape(N)
    cons1 = cons_scale.reshape(1)
    cons16 = jnp.broadcast_to(cons1, (16,))

    aty, ax_tail = _get_sc_matvecs()(AT, A, xf, yf, vsf, csf, cons16)

    sq = (32, 128)
    x2 = x.reshape(sq)
    vs2 = vscale.reshape(sq)
    y2 = y.reshape(sq)
    cs2 = cscale.reshape(sq)
    b2 = b.reshape(sq)
    c2 = c.reshape(sq)
    Iy2 = Iy.reshape(sq)
    il2 = il.reshape(sq)
    iu2 = iu.reshape(sq)
    l2 = l.reshape(sq)
    u2 = u.reshape(sq)

    smem = pl.BlockSpec(memory_space=pltpu.SMEM)
    anyspace = pl.BlockSpec(memory_space=pl.ANY)
    full = pl.BlockSpec(sq, lambda: (0, 0))
    full_col = pl.BlockSpec((N, 1), lambda: (0, 0))

    qx2, parts = pl.pallas_call(
        _tcs_kernel,
        in_specs=[smem, anyspace, anyspace, full_col, full_col]
        + [full] * 11,
        out_specs=(full, smem),
        out_shape=(jax.ShapeDtypeStruct(sq, jnp.float32),
                   jax.ShapeDtypeStruct((9,), jnp.float32)),
        scratch_shapes=[
            pltpu.VMEM((DEPTH, BLK, N), jnp.float32),
            pltpu.SMEM((9,), jnp.float32),
            pltpu.SemaphoreType.DMA((DEPTH,)),
        ],
        compiler_params=pltpu.CompilerParams(
            vmem_limit_bytes=100 * 1024 * 1024),
    )(cons1, Q, A, x, vscale, x2, vs2, y2, cs2, b2, c2,
      Iy2, il2, iu2, l2, u2)

    st = (A_TAIL // 128, 128)
    g0 = lambda: (0, 0)
    out = pl.pallas_call(
        _tc3_kernel,
        in_specs=[smem] * 2
        + [pl.BlockSpec(sq, g0)] * 2
        + [pl.BlockSpec(st, g0)] * 3
        + [pl.BlockSpec(sq, g0)] * 8,
        out_specs=smem,
        out_shape=jax.ShapeDtypeStruct((4,), jnp.float32),
    )(cons1, parts,
      qx2, aty.reshape(sq),
      ax_tail.reshape(st), b2[A_HEAD // 128:], Iy2[A_HEAD // 128:],
      c2, y2, cs2, Iy2, il2, iu2, l2, u2)

    res = out[0].reshape(1, 1)
    t1 = out[1].reshape(())
    t2 = out[2].reshape(())
    t3 = out[3].reshape(1, 1)
    return res, t1, t2, t3


# speedup vs baseline: 1.2243x; 1.2049x over previous
import functools
import jax
import jax.numpy as jnp
from jax import lax
from jax.experimental import pallas as pl
from jax.experimental.pallas import tpu as pltpu
from jax.experimental.pallas import tpu_sc as plsc

N = 4096
A_TAIL = 1536
A_HEAD = N - A_TAIL

NC = 2
NS = 16
NW = NC * NS
AT_W = N // NW
ATAIL_W = A_TAIL // NW
RB = 8
CHUNKS = N // 16
UNROLL = 4


def _compute_block(buf, v_ref, red_v, out_v, blk):
    zero = jnp.zeros((16,), jnp.float32)

    def jbody(j, accs):
        accs = list(accs)
        for u in range(UNROLL):
            sl = pl.ds((j * UNROLL + u) * 16, 16)
            vc = v_ref[sl]
            for r in range(RB):
                accs[r] = accs[r] + buf[r, sl] * vc
        return tuple(accs)

    accs = lax.fori_loop(0, CHUNKS // UNROLL, jbody, (zero,) * RB)
    for r in range(RB):
        red_v[r, :] = accs[r]
    lanes = lax.iota(jnp.int32, 16)
    row_idx = lanes & (RB - 1)
    hsum = zero
    for j in range(16):
        hsum = hsum + plsc.load_gather(
            red_v, [row_idx, jnp.full((16,), j, jnp.int32)])
    plsc.store_scatter(out_v, [blk * RB + row_idx], hsum, mask=lanes < RB)


@functools.cache
def _get_sc_matvecs():
  mesh = plsc.VectorSubcoreMesh(core_axis_name="c", subcore_axis_name="s",
                                num_cores=NC, num_subcores=NS)

  @functools.partial(
      pl.kernel,
      out_type=(jax.ShapeDtypeStruct((N,), jnp.float32),
                jax.ShapeDtypeStruct((A_TAIL,), jnp.float32)),
      mesh=mesh,
      compiler_params=pltpu.CompilerParams(needs_layout_passes=False),
      scratch_types=[
          pltpu.VMEM((N,), jnp.float32),
          pltpu.VMEM((N,), jnp.float32),
          pltpu.VMEM((N,), jnp.float32),
          pltpu.VMEM((RB, N), jnp.float32),
          pltpu.VMEM((RB, N), jnp.float32),
          pltpu.VMEM((AT_W,), jnp.float32),
          pltpu.VMEM((ATAIL_W,), jnp.float32),
          pltpu.VMEM((RB, 16), jnp.float32),
          pltpu.VMEM((16,), jnp.float32),
          pltpu.SemaphoreType.DMA,
          pltpu.SemaphoreType.DMA,
      ],
  )
  def _sc_matvecs(AT_hbm, A_hbm, x_hbm, y_hbm, vs_hbm, cs_hbm, cons_hbm,
                  aty_hbm, axt_hbm,
                  xun_v, yun_v, tmp_v, buf0, buf1, atyv, axtv, red_v, cons_v,
                  sem0, sem1):
    c = lax.axis_index("c")
    s = lax.axis_index("s")
    wid = s * NC + c

    pltpu.sync_copy(cons_hbm, cons_v)
    csv = cons_v[...]

    pltpu.sync_copy(x_hbm, xun_v)
    pltpu.sync_copy(vs_hbm, tmp_v)

    def unscale_x(j, _):
        sl = pl.ds(j * 16, 16)
        xun_v[sl] = xun_v[sl] / tmp_v[sl] * csv
        return 0

    lax.fori_loop(0, CHUNKS, unscale_x, 0)

    pltpu.sync_copy(y_hbm, yun_v)
    pltpu.sync_copy(cs_hbm, tmp_v)

    def unscale_y(j, _):
        sl = pl.ds(j * 16, 16)
        yun_v[sl] = yun_v[sl] / tmp_v[sl] * csv
        return 0

    lax.fori_loop(0, CHUNKS, unscale_y, 0)

    def do_matvec(M_hbm, row0, nrows, v_ref, out_v):
        nblk = nrows // RB
        pltpu.async_copy(M_hbm.at[pl.ds(row0, RB)], buf0, sem0)

        def outer(k, _):
            blk0 = k * 2
            pltpu.async_copy(
                M_hbm.at[pl.ds(row0 + (blk0 + 1) * RB, RB)], buf1, sem1)
            pltpu.make_async_copy(
                M_hbm.at[pl.ds(row0, RB)], buf0, sem0).wait()
            _compute_block(buf0, v_ref, red_v, out_v, blk0)

            @pl.when(blk0 + 2 < nblk)
            def _():
                pltpu.async_copy(
                    M_hbm.at[pl.ds(row0 + (blk0 + 2) * RB, RB)], buf0, sem0)

            pltpu.make_async_copy(
                M_hbm.at[pl.ds(row0, RB)], buf1, sem1).wait()
            _compute_block(buf1, v_ref, red_v, out_v, blk0 + 1)
            return 0

        lax.fori_loop(0, nblk // 2, outer, 0)

    do_matvec(AT_hbm, wid * AT_W, AT_W, yun_v, atyv)
    do_matvec(A_hbm, A_HEAD + wid * ATAIL_W, ATAIL_W, xun_v, axtv)

    pltpu.sync_copy(atyv, aty_hbm.at[pl.ds(wid * AT_W, AT_W)])
    pltpu.sync_copy(axtv, axt_hbm.at[pl.ds(wid * ATAIL_W, ATAIL_W)])

  return _sc_matvecs


BLK = 256
NB_M = N // BLK
NB = 3 * NB_M
DEPTH = 4
R2 = BLK // 128

_VAR, _B, _QX, _C, _QUAD, _LIN, _VIOT, _CV, _AX = range(9)


def _tcs_kernel(cons_ref, Q_ref, A_ref, AT_ref, xc_ref, vsc_ref,
                yc_ref, csc_ref,
                x2_ref, vs2_ref, y2_ref, cs2_ref, b2_ref, c2_ref,
                Iy2_ref, il2_ref, iu2_ref, l2_ref, u2_ref,
                qx2_out, aty2_out, part_ref, qbuf, acc_ref, sems):
    relu = jax.nn.relu
    cs = cons_ref[0]
    xun_c = xc_ref[...] / vsc_ref[...] * cs
    yun_c = yc_ref[...] / csc_ref[...] * cs

    def start_dma(i, slot):
        row = lax.rem(i, NB_M) * BLK

        @pl.when(i < NB_M)
        def _():
            pltpu.make_async_copy(
                Q_ref.at[pl.ds(row, BLK)], qbuf.at[slot], sems.at[slot]
            ).start()

        @pl.when(jnp.logical_and(i >= NB_M, i < 2 * NB_M))
        def _():
            pltpu.make_async_copy(
                A_ref.at[pl.ds(row, BLK)], qbuf.at[slot], sems.at[slot]
            ).start()

        @pl.when(i >= 2 * NB_M)
        def _():
            pltpu.make_async_copy(
                AT_ref.at[pl.ds(row, BLK)], qbuf.at[slot], sems.at[slot]
            ).start()

    for d in range(DEPTH):
        start_dma(d, d)

    for k in range(9):
        acc_ref[k] = 0.0

    def step(i, _):
        slot = lax.rem(i, DEPTH)
        pltpu.make_async_copy(
            Q_ref.at[pl.ds(0, BLK)], qbuf.at[slot], sems.at[slot]
        ).wait()
        mb = qbuf[pl.ds(slot, 1)].reshape(BLK, N)
        sl = pl.ds(lax.rem(i, NB_M) * R2, R2)

        @pl.when(i < 2 * NB_M)
        def _qa():
            mv2 = jnp.dot(mb, xun_c,
                          preferred_element_type=jnp.float32).reshape(R2, 128)

            @pl.when(i < NB_M)
            def _qpart():
                qx2_out[sl, :] = mv2
                xun2 = x2_ref[sl, :] / vs2_ref[sl, :] * cs
                yun2 = y2_ref[sl, :] / cs2_ref[sl, :] * cs
                b2 = b2_ref[sl, :]
                c2 = c2_ref[sl, :]
                var_vio = relu(l2_ref[sl, :] - xun2) * il2_ref[sl, :] + \
                    relu(xun2 - u2_ref[sl, :]) * iu2_ref[sl, :]
                acc_ref[_VAR] = jnp.maximum(acc_ref[_VAR],
                                            jnp.max(jnp.abs(var_vio)))
                acc_ref[_B] = jnp.maximum(acc_ref[_B], jnp.max(jnp.abs(b2)))
                acc_ref[_QX] = jnp.maximum(acc_ref[_QX],
                                           jnp.max(jnp.abs(mv2)))
                acc_ref[_C] = jnp.maximum(acc_ref[_C], jnp.max(jnp.abs(c2)))
                acc_ref[_QUAD] = acc_ref[_QUAD] + jnp.sum(xun2 * mv2)
                acc_ref[_LIN] = acc_ref[_LIN] + jnp.sum(c2 * xun2)
                acc_ref[_VIOT] = acc_ref[_VIOT] + jnp.sum(b2 * yun2)

            @pl.when(i >= NB_M)
            def _apart():
                cv = b2_ref[sl, :] - mv2
                cv = cv + relu(-cv) * Iy2_ref[sl, :]
                acc_ref[_CV] = jnp.maximum(acc_ref[_CV],
                                           jnp.max(jnp.abs(cv)))
                acc_ref[_AX] = jnp.maximum(acc_ref[_AX],
                                           jnp.max(jnp.abs(mv2)))

        @pl.when(i >= 2 * NB_M)
        def _atpart():
            mv2 = jnp.dot(mb, yun_c,
                          preferred_element_type=jnp.float32).reshape(R2, 128)
            aty2_out[sl, :] = mv2

        @pl.when(i + DEPTH < NB)
        def _():
            start_dma(i + DEPTH, slot)

        return 0

    lax.fori_loop(0, NB, step, 0)

    for k in range(9):
        part_ref[k] = acc_ref[k]


def _tc3_kernel(cons_ref, pq_ref,
                qx2_ref, aty2_ref,
                c2_ref, y2_ref, cs2_ref, Iy2_ref,
                il2_ref, iu2_ref, l2_ref, u2_ref, out_ref):
    relu = jax.nn.relu
    cs = cons_ref[0]
    yun = y2_ref[...] / cs2_ref[...] * cs

    Qx = qx2_ref[...]
    ATy = aty2_ref[...]
    c = c2_ref[...]

    pg = c - ATy + Qx
    rpg = relu(pg)
    rng = relu(-pg)
    il = il2_ref[...]
    iu = iu2_ref[...]
    RCV = pg - rpg * il + rng * iu
    DR = relu(-yun) * Iy2_ref[...]
    RC = rpg * il - rng * iu
    tm = jnp.where(RC > 0, l2_ref[...], u2_ref[...])

    t1 = jnp.maximum(pq_ref[_VAR], pq_ref[_CV]) / \
        (1.0 + jnp.maximum(pq_ref[_AX], pq_ref[_B]))

    m_rcv = jnp.maximum(jnp.max(jnp.abs(RCV)), jnp.max(jnp.abs(DR)))
    m_aty = jnp.max(jnp.abs(ATy))
    t2 = m_rcv / (1.0 + jnp.maximum(pq_ref[_QX],
                                    jnp.maximum(m_aty, pq_ref[_C])))

    quad = pq_ref[_QUAD]
    lin = pq_ref[_LIN]
    vio = pq_ref[_VIOT]
    rcc = jnp.sum(RC * tm)
    t3 = jnp.abs(quad + lin - vio - rcc) / (
        1.0 + jnp.maximum(jnp.abs(vio - 0.5 * quad),
                          jnp.abs(0.5 * quad + lin)))
    res = jnp.maximum(t1, jnp.maximum(t2, t3))
    out_ref[0] = res
    out_ref[1] = t1
    out_ref[2] = t2
    out_ref[3] = t3


def kernel(Q, A, AT, b, c, x, y, Iy, il, iu, l, u, vscale, cscale, cons_scale):
    cons1 = cons_scale.reshape(1)

    sq = (32, 128)
    x2 = x.reshape(sq)
    vs2 = vscale.reshape(sq)
    y2 = y.reshape(sq)
    cs2 = cscale.reshape(sq)
    b2 = b.reshape(sq)
    c2 = c.reshape(sq)
    Iy2 = Iy.reshape(sq)
    il2 = il.reshape(sq)
    iu2 = iu.reshape(sq)
    l2 = l.reshape(sq)
    u2 = u.reshape(sq)

    smem = pl.BlockSpec(memory_space=pltpu.SMEM)
    anyspace = pl.BlockSpec(memory_space=pl.ANY)
    full = pl.BlockSpec(sq, lambda: (0, 0))
    full_col = pl.BlockSpec((N, 1), lambda: (0, 0))

    qx2, aty2, parts = pl.pallas_call(
        _tcs_kernel,
        in_specs=[smem, anyspace, anyspace, anyspace,
                  full_col, full_col, full_col, full_col]
        + [full] * 11,
        out_specs=(full, full, smem),
        out_shape=(jax.ShapeDtypeStruct(sq, jnp.float32),
                   jax.ShapeDtypeStruct(sq, jnp.float32),
                   jax.ShapeDtypeStruct((9,), jnp.float32)),
        scratch_shapes=[
            pltpu.VMEM((DEPTH, BLK, N), jnp.float32),
            pltpu.SMEM((9,), jnp.float32),
            pltpu.SemaphoreType.DMA((DEPTH,)),
        ],
        compiler_params=pltpu.CompilerParams(
            vmem_limit_bytes=100 * 1024 * 1024),
    )(cons1, Q, A, AT, x, vscale, y, cscale,
      x2, vs2, y2, cs2, b2, c2, Iy2, il2, iu2, l2, u2)

    g0 = lambda: (0, 0)
    out = pl.pallas_call(
        _tc3_kernel,
        in_specs=[smem] * 2 + [pl.BlockSpec(sq, g0)] * 10,
        out_specs=smem,
        out_shape=jax.ShapeDtypeStruct((4,), jnp.float32),
    )(cons1, parts, qx2, aty2,
      c2, y2, cs2, Iy2, il2, iu2, l2, u2)

    res = out[0].reshape(1, 1)
    t1 = out[1].reshape(())
    t2 = out[2].reshape(())
    t3 = out[3].reshape(1, 1)
    return res, t1, t2, t3


# speedup vs baseline: 1.5079x; 1.2317x over previous
import functools
import jax
import jax.numpy as jnp
from jax import lax
from jax.experimental import pallas as pl
from jax.experimental.pallas import tpu as pltpu
from jax.experimental.pallas import tpu_sc as plsc

N = 4096
A_TAIL = 1536
A_HEAD = N - A_TAIL

NC = 2
NS = 16
NW = NC * NS
AT_W = N // NW
ATAIL_W = A_TAIL // NW
RB = 8
CHUNKS = N // 16
UNROLL = 4


def _compute_block(buf, v_ref, red_v, out_v, blk):
    zero = jnp.zeros((16,), jnp.float32)

    def jbody(j, accs):
        accs = list(accs)
        for u in range(UNROLL):
            sl = pl.ds((j * UNROLL + u) * 16, 16)
            vc = v_ref[sl]
            for r in range(RB):
                accs[r] = accs[r] + buf[r, sl] * vc
        return tuple(accs)

    accs = lax.fori_loop(0, CHUNKS // UNROLL, jbody, (zero,) * RB)
    for r in range(RB):
        red_v[r, :] = accs[r]
    lanes = lax.iota(jnp.int32, 16)
    row_idx = lanes & (RB - 1)
    hsum = zero
    for j in range(16):
        hsum = hsum + plsc.load_gather(
            red_v, [row_idx, jnp.full((16,), j, jnp.int32)])
    plsc.store_scatter(out_v, [blk * RB + row_idx], hsum, mask=lanes < RB)


@functools.cache
def _get_sc_matvecs():
  mesh = plsc.VectorSubcoreMesh(core_axis_name="c", subcore_axis_name="s",
                                num_cores=NC, num_subcores=NS)

  @functools.partial(
      pl.kernel,
      out_type=(jax.ShapeDtypeStruct((N,), jnp.float32),
                jax.ShapeDtypeStruct((A_TAIL,), jnp.float32)),
      mesh=mesh,
      compiler_params=pltpu.CompilerParams(needs_layout_passes=False),
      scratch_types=[
          pltpu.VMEM((N,), jnp.float32),
          pltpu.VMEM((N,), jnp.float32),
          pltpu.VMEM((N,), jnp.float32),
          pltpu.VMEM((RB, N), jnp.float32),
          pltpu.VMEM((RB, N), jnp.float32),
          pltpu.VMEM((AT_W,), jnp.float32),
          pltpu.VMEM((ATAIL_W,), jnp.float32),
          pltpu.VMEM((RB, 16), jnp.float32),
          pltpu.VMEM((16,), jnp.float32),
          pltpu.SemaphoreType.DMA,
          pltpu.SemaphoreType.DMA,
      ],
  )
  def _sc_matvecs(AT_hbm, A_hbm, x_hbm, y_hbm, vs_hbm, cs_hbm, cons_hbm,
                  aty_hbm, axt_hbm,
                  xun_v, yun_v, tmp_v, buf0, buf1, atyv, axtv, red_v, cons_v,
                  sem0, sem1):
    c = lax.axis_index("c")
    s = lax.axis_index("s")
    wid = s * NC + c

    pltpu.sync_copy(cons_hbm, cons_v)
    csv = cons_v[...]

    pltpu.sync_copy(x_hbm, xun_v)
    pltpu.sync_copy(vs_hbm, tmp_v)

    def unscale_x(j, _):
        sl = pl.ds(j * 16, 16)
        xun_v[sl] = xun_v[sl] / tmp_v[sl] * csv
        return 0

    lax.fori_loop(0, CHUNKS, unscale_x, 0)

    pltpu.sync_copy(y_hbm, yun_v)
    pltpu.sync_copy(cs_hbm, tmp_v)

    def unscale_y(j, _):
        sl = pl.ds(j * 16, 16)
        yun_v[sl] = yun_v[sl] / tmp_v[sl] * csv
        return 0

    lax.fori_loop(0, CHUNKS, unscale_y, 0)

    def do_matvec(M_hbm, row0, nrows, v_ref, out_v):
        nblk = nrows // RB
        pltpu.async_copy(M_hbm.at[pl.ds(row0, RB)], buf0, sem0)

        def outer(k, _):
            blk0 = k * 2
            pltpu.async_copy(
                M_hbm.at[pl.ds(row0 + (blk0 + 1) * RB, RB)], buf1, sem1)
            pltpu.make_async_copy(
                M_hbm.at[pl.ds(row0, RB)], buf0, sem0).wait()
            _compute_block(buf0, v_ref, red_v, out_v, blk0)

            @pl.when(blk0 + 2 < nblk)
            def _():
                pltpu.async_copy(
                    M_hbm.at[pl.ds(row0 + (blk0 + 2) * RB, RB)], buf0, sem0)

            pltpu.make_async_copy(
                M_hbm.at[pl.ds(row0, RB)], buf1, sem1).wait()
            _compute_block(buf1, v_ref, red_v, out_v, blk0 + 1)
            return 0

        lax.fori_loop(0, nblk // 2, outer, 0)

    do_matvec(AT_hbm, wid * AT_W, AT_W, yun_v, atyv)
    do_matvec(A_hbm, A_HEAD + wid * ATAIL_W, ATAIL_W, xun_v, axtv)

    pltpu.sync_copy(atyv, aty_hbm.at[pl.ds(wid * AT_W, AT_W)])
    pltpu.sync_copy(axtv, axt_hbm.at[pl.ds(wid * ATAIL_W, ATAIL_W)])

  return _sc_matvecs


BLK = 256
NB_M = N // BLK
NB = 3 * NB_M
DEPTH = 4
R2 = BLK // 128

_VAR, _B, _QX, _C, _QUAD, _LIN, _VIOT, _CV, _AX = range(9)


def _tcs_kernel(cons_ref, Q_ref, A_ref, AT_ref, xr_ref, vsr_ref,
                yr_ref, csr_ref,
                x2_ref, vs2_ref, y2_ref, cs2_ref, b2_ref, c2_ref,
                Iy2_ref, il2_ref, iu2_ref, l2_ref, u2_ref,
                out_ref, qx2_out, aty2_out, qbuf, acc_ref, sems):
    relu = jax.nn.relu
    cs = cons_ref[0]
    xun_r = xr_ref[...] / vsr_ref[...] * cs
    yun_r = yr_ref[...] / csr_ref[...] * cs
    _dn = (((1,), (1,)), ((), ()))

    def start_dma(i, slot):
        row = lax.rem(i, NB_M) * BLK

        @pl.when(i < NB_M)
        def _():
            pltpu.make_async_copy(
                Q_ref.at[pl.ds(row, BLK)], qbuf.at[slot], sems.at[slot]
            ).start()

        @pl.when(jnp.logical_and(i >= NB_M, i < 2 * NB_M))
        def _():
            pltpu.make_async_copy(
                A_ref.at[pl.ds(row, BLK)], qbuf.at[slot], sems.at[slot]
            ).start()

        @pl.when(i >= 2 * NB_M)
        def _():
            pltpu.make_async_copy(
                AT_ref.at[pl.ds(row, BLK)], qbuf.at[slot], sems.at[slot]
            ).start()

    for d in range(DEPTH):
        start_dma(d, d)

    for k in range(9):
        acc_ref[k] = 0.0

    def step(i, _):
        slot = lax.rem(i, DEPTH)
        pltpu.make_async_copy(
            Q_ref.at[pl.ds(0, BLK)], qbuf.at[slot], sems.at[slot]
        ).wait()
        mb = qbuf[pl.ds(slot, 1)].reshape(BLK, N)
        sl = pl.ds(lax.rem(i, NB_M) * R2, R2)

        @pl.when(i < 2 * NB_M)
        def _qa():
            mv2 = lax.dot_general(
                mb, xun_r, _dn,
                preferred_element_type=jnp.float32).reshape(R2, 128)

            @pl.when(i < NB_M)
            def _qpart():
                qx2_out[sl, :] = mv2
                xun2 = x2_ref[sl, :] / vs2_ref[sl, :] * cs
                yun2 = y2_ref[sl, :] / cs2_ref[sl, :] * cs
                b2 = b2_ref[sl, :]
                c2 = c2_ref[sl, :]
                var_vio = relu(l2_ref[sl, :] - xun2) * il2_ref[sl, :] + \
                    relu(xun2 - u2_ref[sl, :]) * iu2_ref[sl, :]
                acc_ref[_VAR] = jnp.maximum(acc_ref[_VAR],
                                            jnp.max(jnp.abs(var_vio)))
                acc_ref[_B] = jnp.maximum(acc_ref[_B], jnp.max(jnp.abs(b2)))
                acc_ref[_QX] = jnp.maximum(acc_ref[_QX],
                                           jnp.max(jnp.abs(mv2)))
                acc_ref[_C] = jnp.maximum(acc_ref[_C], jnp.max(jnp.abs(c2)))
                acc_ref[_QUAD] = acc_ref[_QUAD] + jnp.sum(xun2 * mv2)
                acc_ref[_LIN] = acc_ref[_LIN] + jnp.sum(c2 * xun2)
                acc_ref[_VIOT] = acc_ref[_VIOT] + jnp.sum(b2 * yun2)

            @pl.when(i >= NB_M)
            def _apart():
                cv = b2_ref[sl, :] - mv2
                cv = cv + relu(-cv) * Iy2_ref[sl, :]
                acc_ref[_CV] = jnp.maximum(acc_ref[_CV],
                                           jnp.max(jnp.abs(cv)))
                acc_ref[_AX] = jnp.maximum(acc_ref[_AX],
                                           jnp.max(jnp.abs(mv2)))

        @pl.when(i >= 2 * NB_M)
        def _atpart():
            mv2 = lax.dot_general(
                mb, yun_r, _dn,
                preferred_element_type=jnp.float32).reshape(R2, 128)
            aty2_out[sl, :] = mv2

        @pl.when(i + DEPTH < NB)
        def _():
            start_dma(i + DEPTH, slot)

        return 0

    lax.fori_loop(0, NB, step, 0)

    yun = y2_ref[...] / cs2_ref[...] * cs
    Qx = qx2_out[...]
    ATy = aty2_out[...]
    c = c2_ref[...]
    pg = c - ATy + Qx
    rpg = relu(pg)
    rng = relu(-pg)
    il = il2_ref[...]
    iu = iu2_ref[...]
    RCV = pg - rpg * il + rng * iu
    DR = relu(-yun) * Iy2_ref[...]
    RC = rpg * il - rng * iu
    tm = jnp.where(RC > 0, l2_ref[...], u2_ref[...])

    t1 = jnp.maximum(acc_ref[_VAR], acc_ref[_CV]) / \
        (1.0 + jnp.maximum(acc_ref[_AX], acc_ref[_B]))
    m_rcv = jnp.maximum(jnp.max(jnp.abs(RCV)), jnp.max(jnp.abs(DR)))
    m_aty = jnp.max(jnp.abs(ATy))
    t2 = m_rcv / (1.0 + jnp.maximum(acc_ref[_QX],
                                    jnp.maximum(m_aty, acc_ref[_C])))
    quad = acc_ref[_QUAD]
    lin = acc_ref[_LIN]
    vio = acc_ref[_VIOT]
    rcc = jnp.sum(RC * tm)
    t3 = jnp.abs(quad + lin - vio - rcc) / (
        1.0 + jnp.maximum(jnp.abs(vio - 0.5 * quad),
                          jnp.abs(0.5 * quad + lin)))
    res = jnp.maximum(t1, jnp.maximum(t2, t3))
    out_ref[0] = res
    out_ref[1] = t1
    out_ref[2] = t2
    out_ref[3] = t3


def _tc3_kernel(cons_ref, pq_ref,
                qx2_ref, aty2_ref,
                c2_ref, y2_ref, cs2_ref, Iy2_ref,
                il2_ref, iu2_ref, l2_ref, u2_ref, out_ref):
    relu = jax.nn.relu
    cs = cons_ref[0]
    yun = y2_ref[...] / cs2_ref[...] * cs

    Qx = qx2_ref[...]
    ATy = aty2_ref[...]
    c = c2_ref[...]

    pg = c - ATy + Qx
    rpg = relu(pg)
    rng = relu(-pg)
    il = il2_ref[...]
    iu = iu2_ref[...]
    RCV = pg - rpg * il + rng * iu
    DR = relu(-yun) * Iy2_ref[...]
    RC = rpg * il - rng * iu
    tm = jnp.where(RC > 0, l2_ref[...], u2_ref[...])

    t1 = jnp.maximum(pq_ref[_VAR], pq_ref[_CV]) / \
        (1.0 + jnp.maximum(pq_ref[_AX], pq_ref[_B]))

    m_rcv = jnp.maximum(jnp.max(jnp.abs(RCV)), jnp.max(jnp.abs(DR)))
    m_aty = jnp.max(jnp.abs(ATy))
    t2 = m_rcv / (1.0 + jnp.maximum(pq_ref[_QX],
                                    jnp.maximum(m_aty, pq_ref[_C])))

    quad = pq_ref[_QUAD]
    lin = pq_ref[_LIN]
    vio = pq_ref[_VIOT]
    rcc = jnp.sum(RC * tm)
    t3 = jnp.abs(quad + lin - vio - rcc) / (
        1.0 + jnp.maximum(jnp.abs(vio - 0.5 * quad),
                          jnp.abs(0.5 * quad + lin)))
    res = jnp.maximum(t1, jnp.maximum(t2, t3))
    out_ref[0] = res
    out_ref[1] = t1
    out_ref[2] = t2
    out_ref[3] = t3


def kernel(Q, A, AT, b, c, x, y, Iy, il, iu, l, u, vscale, cscale, cons_scale):
    cons1 = cons_scale.reshape(1)

    sq = (32, 128)
    rw = (1, N)
    x2 = x.reshape(sq)
    vs2 = vscale.reshape(sq)
    y2 = y.reshape(sq)
    cs2 = cscale.reshape(sq)
    b2 = b.reshape(sq)
    c2 = c.reshape(sq)
    Iy2 = Iy.reshape(sq)
    il2 = il.reshape(sq)
    iu2 = iu.reshape(sq)
    l2 = l.reshape(sq)
    u2 = u.reshape(sq)

    smem = pl.BlockSpec(memory_space=pltpu.SMEM)
    anyspace = pl.BlockSpec(memory_space=pl.ANY)
    full = pl.BlockSpec(sq, lambda: (0, 0))
    full_row = pl.BlockSpec(rw, lambda: (0, 0))

    out = pl.pallas_call(
        _tcs_kernel,
        in_specs=[smem, anyspace, anyspace, anyspace,
                  full_row, full_row, full_row, full_row]
        + [full] * 11,
        out_specs=smem,
        out_shape=jax.ShapeDtypeStruct((4,), jnp.float32),
        scratch_shapes=[
            pltpu.VMEM(sq, jnp.float32),
            pltpu.VMEM(sq, jnp.float32),
            pltpu.VMEM((DEPTH, BLK, N), jnp.float32),
            pltpu.SMEM((9,), jnp.float32),
            pltpu.SemaphoreType.DMA((DEPTH,)),
        ],
        compiler_params=pltpu.CompilerParams(
            vmem_limit_bytes=100 * 1024 * 1024),
    )(cons1, Q, A, AT,
      x.reshape(rw), vscale.reshape(rw), y.reshape(rw), cscale.reshape(rw),
      x2, vs2, y2, cs2, b2, c2, Iy2, il2, iu2, l2, u2)

    res = out[0].reshape(1, 1)
    t1 = out[1].reshape(())
    t2 = out[2].reshape(())
    t3 = out[3].reshape(1, 1)
    return res, t1, t2, t3
